# baseline (speedup 1.0000x reference)
import functools
import math

import jax
import jax.numpy as jnp
from jax import lax
from jax.experimental import pallas as pl
from jax.experimental.pallas import tpu as pltpu
from jax.experimental.pallas import tpu_sc as plsc

NA = 10000
K = 16
NC = 100
P = 16
M = 16
CUT = 5.0
ETA = 4.0
H = 128
E = NA * K
B = 200
BK = B * K
GRID = NA // B
PI_C = math.pi / CUT
C0 = math.sqrt(2.0 / CUT)


def _tc_body(x_ref, xj_ref, t_ref, W1_ref, b1_ref, W2_ref, b2_ref, W3_ref,
             b3_ref, W3t_ref, W2t_ref, W1t_ref, fi_ref, g_ref):
    f32 = jnp.float32
    a = (jax.lax.broadcasted_iota(jnp.int32, (1, P), 1).astype(f32)
         + 1.0) * PI_C
    xi = x_ref[...]
    xj = xj_ref[..., :3]
    t = t_ref[...]
    xi_rep = jnp.broadcast_to(xi[:, None, :], (B, K, 3)).reshape(BK, 3)
    diff = xi_rep - t - xj
    r2 = jnp.sum(diff * diff, axis=1, keepdims=True)
    r = jnp.sqrt(r2)
    rinv = 1.0 / jnp.maximum(r, 1e-12)
    u = diff * rinv
    ar = a * r
    q = ar * (1.0 / (2.0 * math.pi)) + 0.5
    k = q.astype(jnp.int32).astype(f32)
    xr = ar - k * (2.0 * math.pi)
    x2 = xr * xr
    s_ar = xr * (9.9999995621e-01 + x2 * (-1.6666631907e-01 + x2 * (
        8.3328905394e-03 + x2 * (-1.9820753685e-04 + x2 * (
            2.7127960656e-06 + x2 * -2.0872484296e-08)))))
    c_ar = 9.9999999228e-01 + x2 * (-4.9999991770e-01 + x2 * (
        4.1666524306e-02 + x2 * (-1.3887970098e-03 + x2 * (
            2.4773416988e-05 + x2 * (-2.7113298104e-07
                                     + x2 * 1.7368844696e-09)))))
    fc = 0.5 * (c_ar[:, 0:1] + 1.0)
    s1 = s_ar[:, 0:1]
    sb = C0 * s_ar * rinv
    rbf = fc * sb

    KK = K * K
    rio = jax.lax.broadcasted_iota(jnp.int32, (K, KK), 0)
    cio = jax.lax.broadcasted_iota(jnp.int32, (K, KK), 1)
    RepMat = (rio == jax.lax.shift_right_logical(cio, 4)).astype(f32)
    TileMat = (rio == (cio & 15)).astype(f32)
    rio2 = jax.lax.broadcasted_iota(jnp.int32, (KK, K), 0)
    cio2 = jax.lax.broadcasted_iota(jnp.int32, (KK, K), 1)
    SumK = (jax.lax.shift_right_logical(rio2, 4) == cio2).astype(f32)
    SumL = ((rio2 & 15) == cio2).astype(f32)
    cl = jax.lax.broadcasted_iota(jnp.int32, (1, KK), 1)
    maskL = (jax.lax.shift_right_logical(cl, 4) != (cl & 15)).astype(f32)
    rwio = jax.lax.broadcasted_iota(jnp.int32, (BK, KK), 0)
    cwio = jax.lax.broadcasted_iota(jnp.int32, (BK, KK), 1)
    SEL2 = ((rwio & 15) == (cwio & 15)).astype(f32)

    def dotf(aa, bb):
        return jnp.dot(aa, bb, preferred_element_type=f32)

    rio4 = jax.lax.broadcasted_iota(jnp.int32, (4, 4 * K), 0)
    cio4 = jax.lax.broadcasted_iota(jnp.int32, (4, 4 * K), 1)
    Rep4 = (rio4 == jax.lax.shift_right_logical(cio4, 4)).astype(f32)
    rio5 = jax.lax.broadcasted_iota(jnp.int32, (4 * K, 4), 0)
    cio5 = jax.lax.broadcasted_iota(jnp.int32, (4 * K, 4), 1)
    Sum4 = (jax.lax.shift_right_logical(rio5, 4) == cio5).astype(f32)
    rio6 = jax.lax.broadcasted_iota(jnp.int32, (BK, 4 * K), 0)
    cio6 = jax.lax.broadcasted_iota(jnp.int32, (BK, 4 * K), 1)
    SEL64 = ((rio6 & 15) == (cio6 & 15)).astype(f32)

    in4 = jnp.concatenate([fc, u], axis=1)
    Z = dotf(in4, Rep4) * SEL64
    A64 = jnp.sum(Z.reshape(B, K, 4 * K), axis=1)
    fcL16 = A64[:, 0:K]
    uL16 = [A64[:, K + K * d:2 * K + K * d] for d in range(3)]
    wgt = dotf(fcL16, TileMat) * maskL
    uA = [dotf(uL16[d], RepMat) for d in range(3)]
    uB = [dotf(uL16[d], TileMat) for d in range(3)]
    cos = uA[0] * uB[0] + uA[1] * uB[1] + uA[2] * uB[2]

    Gms = []
    Dm16s = []
    for m in range(M):
        mu_m = -1.0 + 2.0 * m / (M - 1)
        Gm = jnp.exp(-ETA * (cos - mu_m) ** 2)
        Gms.append(Gm)
        Dm16s.append(dotf(Gm * wgt, SumK))
    Dmat = jnp.concatenate(Dm16s, axis=1)
    Dr = jnp.broadcast_to(Dmat[:, None, :], (B, K, KK)).reshape(BK, KK)
    DP = dotf(Dr * SEL2, SumK)
    desc = jnp.concatenate([rbf, DP], axis=1)

    z1 = jnp.dot(desc, W1_ref[...], preferred_element_type=f32) + b1_ref[...]
    h1 = jax.nn.softplus(z1)
    z2 = jnp.dot(h1, W2_ref[...], preferred_element_type=f32) + b2_ref[...]
    h2 = jax.nn.softplus(z2)
    pre = jnp.dot(h2, W3_ref[...], preferred_element_type=f32) + b3_ref[...]
    e = pre * fc

    dfc = pre
    dh2 = fc * W3t_ref[...]
    dz2 = dh2 * jax.nn.sigmoid(z2)
    dh1 = jnp.dot(dz2, W2t_ref[...], preferred_element_type=f32)
    dz1 = dh1 * jax.nn.sigmoid(z1)
    ddesc = jnp.dot(dz1, W1t_ref[...], preferred_element_type=f32)
    drbf = ddesc[:, :P]
    dD = ddesc[:, P:]

    rio3 = jax.lax.broadcasted_iota(jnp.int32, (2 * P, 2), 0)
    cio3 = jax.lax.broadcasted_iota(jnp.int32, (2 * P, 2), 1)
    RED = (jax.lax.shift_right_logical(rio3, 4) == cio3).astype(f32)
    dsb = drbf * fc
    red = dotf(jnp.concatenate(
        [drbf * sb, dsb * (C0 * a * c_ar - sb)], axis=1), RED)
    dfc = dfc + red[:, 0:1]
    dr = red[:, 1:2] * rinv

    dDz = dotf(dD, RepMat) * SEL2
    dDatm = jnp.sum(dDz.reshape(B, K, KK), axis=1)
    S0 = None
    S1 = None
    for m in range(M):
        mu_m = -1.0 + 2.0 * m / (M - 1)
        Hm = dotf(dDatm[:, m * K:(m + 1) * K], RepMat) * Gms[m]
        S0 = Hm if S0 is None else S0 + Hm
        S1 = mu_m * Hm if S1 is None else S1 + mu_m * Hm
    dfcD16 = dotf(S0 * maskL, SumL)
    dcos = (-2.0 * ETA) * (cos * S0 - S1) * wgt
    du16s = [dotf(dcos * uB[d], SumK) + dotf(dcos * uA[d], SumL)
             for d in range(3)]
    Xcat = jnp.concatenate([dfcD16] + du16s, axis=1)
    Xr = jnp.broadcast_to(Xcat[:, None, :], (B, K, 4 * K)).reshape(BK, 4 * K)
    out4 = dotf(Xr * SEL64, Sum4)
    dfc = dfc + out4[:, 0:1]
    du = out4[:, 1:4]

    dr = dr + dfc * (-0.5 * PI_C) * s1
    ddiff = dr * u + (du - jnp.sum(du * u, axis=1, keepdims=True) * u) * rinv

    fi_ref[...] = -jnp.sum(ddiff.reshape(B, K, 3), axis=1)
    g_ref[...] = jnp.concatenate(
        [ddiff, e, jnp.zeros((BK, 12), f32)], axis=1)


@functools.partial(jax.jit, static_argnames=("interpret",))
def _tc_call(x, xj16, transform_x, W1, b1, W2, b2, W3, b3, interpret=False):
    rep = lambda i: (0, 0)
    return pl.pallas_call(
        _tc_body,
        grid=(GRID,),
        in_specs=[
            pl.BlockSpec((B, 3), lambda i: (i, 0)),
            pl.BlockSpec((BK, 16), lambda i: (i, 0)),
            pl.BlockSpec((BK, 3), lambda i: (i, 0)),
            pl.BlockSpec((P + M, H), rep),
            pl.BlockSpec((1, H), rep),
            pl.BlockSpec((H, H), rep),
            pl.BlockSpec((1, H), rep),
            pl.BlockSpec((H, 1), rep),
            pl.BlockSpec((1, 1), rep),
            pl.BlockSpec((1, H), rep),
            pl.BlockSpec((H, H), rep),
            pl.BlockSpec((H, P + M), rep),
        ],
        out_specs=[
            pl.BlockSpec((B, 3), lambda i: (i, 0)),
            pl.BlockSpec((BK, 16), lambda i: (i, 0)),
        ],
        out_shape=[
            jax.ShapeDtypeStruct((NA, 3), jnp.float32),
            jax.ShapeDtypeStruct((E, 16), jnp.float32),
        ],
        interpret=interpret,
    )(x, xj16, transform_x, W1, b1.reshape(1, H), W2, b2.reshape(1, H), W3,
      b3.reshape(1, 1), W3.reshape(1, H), W2.T, W1.T)


NW = 32
CH = 100
NCH = E // (NW * CH)
RPS = NA // 16

def _sc_gather_body(x16_hbm, uj_hbm, out_hbm, idx_v, rows_v, sem):
    w = lax.axis_index("c") * 16 + lax.axis_index("s")
    pltpu.sync_copy(uj_hbm.at[w], idx_v)

    def body(j, carry):
        pltpu.async_copy(x16_hbm.at[idx_v.at[j]], rows_v.at[j], sem).wait()
        return carry

    lax.fori_loop(0, NCH, body, 0)
    pltpu.sync_copy(rows_v, out_hbm.at[w])


def _sc_scatter_body(g_hbm, uj_hbm, ind_hbm, zf_hbm, ze_hbm, outf_hbm,
                     oute_hbm, idxj_v, idxe_v, vals_v, accf_s, acce_s):
    c = lax.axis_index("c")
    s = lax.axis_index("s")
    w = c * 16 + s
    pltpu.sync_copy(zf_hbm.at[pl.ds(s * RPS, RPS)],
                    accf_s.at[pl.ds(s * RPS, RPS)])

    @pl.when(s == 0)
    def _():
        pltpu.sync_copy(ze_hbm, acce_s)

    plsc.subcore_barrier()
    pltpu.sync_copy(uj_hbm.at[w], idxj_v)
    pltpu.sync_copy(ind_hbm.at[w], idxe_v)
    pltpu.sync_copy(g_hbm.at[w], vals_v)

    def body(j, carry):
        pltpu.sync_copy(vals_v.at[j], accf_s.at[idxj_v.at[j]], add=True)
        pltpu.sync_copy(vals_v.at[j], acce_s.at[idxe_v.at[j]], add=True)
        return carry

    lax.fori_loop(0, NCH, body, 0)
    plsc.subcore_barrier()
    pltpu.sync_copy(accf_s.at[pl.ds(s * RPS, RPS)],
                    outf_hbm.at[c, pl.ds(s * RPS, RPS)])

    @pl.when(s == 0)
    def _():
        pltpu.sync_copy(acce_s, oute_hbm.at[c])


@functools.cache
def _sc_kernels():
    mesh = plsc.VectorSubcoreMesh(core_axis_name="c", subcore_axis_name="s")
    gather = pl.kernel(
        _sc_gather_body,
        out_type=jax.ShapeDtypeStruct((NW, NCH, CH, 16), jnp.float32),
        compiler_params=pltpu.CompilerParams(use_tc_tiling_on_sc=False),
        mesh=mesh,
        scratch_types=[
            pltpu.VMEM((NCH, CH), jnp.int32),
            pltpu.VMEM((NCH, CH, 16), jnp.float32),
            pltpu.SemaphoreType.DMA,
        ],
    )
    scatter = pl.kernel(
        _sc_scatter_body,
        out_type=(jax.ShapeDtypeStruct((2, NA, 16), jnp.float32),
                  jax.ShapeDtypeStruct((2, NC, 16), jnp.float32)),
        compiler_params=pltpu.CompilerParams(use_tc_tiling_on_sc=False),
        mesh=mesh,
        scratch_types=[
            pltpu.VMEM((NCH, CH), jnp.int32),
            pltpu.VMEM((NCH, CH), jnp.int32),
            pltpu.VMEM((NCH, CH, 16), jnp.float32),
            pltpu.VMEM_SHARED((NA, 16), jnp.float32),
            pltpu.VMEM_SHARED((NC, 16), jnp.float32),
        ],
    )
    return gather, scatter


def kernel(x, neighlist, transform_x, indices, atoms_per_structure, types,
           unique_i, unique_j, W1, b1, W2, b2, W3, b3):
    x16 = jnp.pad(x, ((0, 0), (0, 13)))
    ujc = unique_j.astype(jnp.int32).reshape(NW, NCH, CH)
    _sc_gather, _sc_scatter = _sc_kernels()
    xj16 = _sc_gather(x16, ujc).reshape(E, 16)
    fi, g16 = _tc_call(x, xj16, transform_x, W1, b1, W2, b2, W3, b3)
    outf, oute = _sc_scatter(
        g16.reshape(NW, NCH, CH, 16), ujc,
        indices.astype(jnp.int32).reshape(NW, NCH, CH),
        jnp.zeros((NA, 16), jnp.float32), jnp.zeros((NC, 16), jnp.float32))
    energy = oute[0, :, 3] + oute[1, :, 3]
    forces = fi + outf[0, :, :3] + outf[1, :, :3]
    return energy, forces


# speedup vs baseline: 1.1145x; 1.1145x over previous
import functools
import math

import jax
import jax.numpy as jnp
from jax import lax
from jax.experimental import pallas as pl
from jax.experimental.pallas import tpu as pltpu
from jax.experimental.pallas import tpu_sc as plsc

NA = 10000
K = 16
NC = 100
P = 16
M = 16
CUT = 5.0
ETA = 4.0
H = 128
E = NA * K
B = 200
BK = B * K
GRID = NA // B
PI_C = math.pi / CUT
C0 = math.sqrt(2.0 / CUT)


def _tc_body(x_ref, xj_ref, t_ref, W1_ref, b1_ref, W2_ref, b2_ref, W3_ref,
             b3_ref, W3t_ref, W2t_ref, W1t_ref, fi_ref, g_ref):
    f32 = jnp.float32
    a = (jax.lax.broadcasted_iota(jnp.int32, (1, P), 1).astype(f32)
         + 1.0) * PI_C
    xi = x_ref[...]
    xj = xj_ref[..., :3]
    t = t_ref[...]
    xi_rep = jnp.broadcast_to(xi[:, None, :], (B, K, 3)).reshape(BK, 3)
    diff = xi_rep - t - xj
    r2 = jnp.sum(diff * diff, axis=1, keepdims=True)
    r = jnp.sqrt(r2)
    rinv = 1.0 / jnp.maximum(r, 1e-12)
    u = diff * rinv
    ar = a * r
    q = ar * (1.0 / (2.0 * math.pi)) + 0.5
    k = q.astype(jnp.int32).astype(f32)
    xr = ar - k * (2.0 * math.pi)
    x2 = xr * xr
    s_ar = xr * (9.9999995621e-01 + x2 * (-1.6666631907e-01 + x2 * (
        8.3328905394e-03 + x2 * (-1.9820753685e-04 + x2 * (
            2.7127960656e-06 + x2 * -2.0872484296e-08)))))
    c_ar = 9.9999999228e-01 + x2 * (-4.9999991770e-01 + x2 * (
        4.1666524306e-02 + x2 * (-1.3887970098e-03 + x2 * (
            2.4773416988e-05 + x2 * (-2.7113298104e-07
                                     + x2 * 1.7368844696e-09)))))
    fc = 0.5 * (c_ar[:, 0:1] + 1.0)
    s1 = s_ar[:, 0:1]
    sb = C0 * s_ar * rinv
    rbf = fc * sb

    KK = K * K
    rio = jax.lax.broadcasted_iota(jnp.int32, (K, KK), 0)
    cio = jax.lax.broadcasted_iota(jnp.int32, (K, KK), 1)
    RepMat = (rio == jax.lax.shift_right_logical(cio, 4)).astype(f32)
    TileMat = (rio == (cio & 15)).astype(f32)
    rio2 = jax.lax.broadcasted_iota(jnp.int32, (KK, K), 0)
    cio2 = jax.lax.broadcasted_iota(jnp.int32, (KK, K), 1)
    SumK = (jax.lax.shift_right_logical(rio2, 4) == cio2).astype(f32)
    SumL = ((rio2 & 15) == cio2).astype(f32)
    cl = jax.lax.broadcasted_iota(jnp.int32, (1, KK), 1)
    maskL = (jax.lax.shift_right_logical(cl, 4) != (cl & 15)).astype(f32)
    rwio = jax.lax.broadcasted_iota(jnp.int32, (BK, KK), 0)
    cwio = jax.lax.broadcasted_iota(jnp.int32, (BK, KK), 1)
    SEL2 = ((rwio & 15) == (cwio & 15)).astype(f32)

    def dotf(aa, bb):
        return jnp.dot(aa, bb, preferred_element_type=f32)

    sel = (jax.lax.rem(jax.lax.broadcasted_iota(jnp.int32, (BK, K), 0), K)
           == jax.lax.broadcasted_iota(jnp.int32, (BK, K), 1)).astype(f32)

    def pairs_to_lanes(v):
        return jnp.sum((v * sel).reshape(B, K, K), axis=1)

    def lanes_to_pairs(A):
        Ar = jnp.broadcast_to(A[:, None, :], (B, K, K)).reshape(BK, K)
        return jnp.sum(Ar * sel, axis=1, keepdims=True)

    fcL16 = pairs_to_lanes(fc)
    uL16 = [pairs_to_lanes(u[:, d:d + 1]) for d in range(3)]
    wgt = dotf(fcL16, TileMat) * maskL
    uA = [dotf(uL16[d], RepMat) for d in range(3)]
    uB = [dotf(uL16[d], TileMat) for d in range(3)]
    cos = uA[0] * uB[0] + uA[1] * uB[1] + uA[2] * uB[2]

    Gms = []
    Dm16s = []
    for m in range(M):
        mu_m = -1.0 + 2.0 * m / (M - 1)
        Gm = jnp.exp(-ETA * (cos - mu_m) ** 2)
        Gms.append(Gm)
        Dm16s.append(dotf(Gm * wgt, SumK))
    Dmat = jnp.concatenate(Dm16s, axis=1)
    Dr = jnp.broadcast_to(Dmat[:, None, :], (B, K, KK)).reshape(BK, KK)
    DP = dotf(Dr * SEL2, SumK)
    desc = jnp.concatenate([rbf, DP], axis=1)

    z1 = jnp.dot(desc, W1_ref[...], preferred_element_type=f32) + b1_ref[...]
    h1 = jax.nn.softplus(z1)
    z2 = jnp.dot(h1, W2_ref[...], preferred_element_type=f32) + b2_ref[...]
    h2 = jax.nn.softplus(z2)
    pre = jnp.dot(h2, W3_ref[...], preferred_element_type=f32) + b3_ref[...]
    e = pre * fc

    dfc = pre
    dh2 = fc * W3t_ref[...]
    dz2 = dh2 * jax.nn.sigmoid(z2)
    dh1 = jnp.dot(dz2, W2t_ref[...], preferred_element_type=f32)
    dz1 = dh1 * jax.nn.sigmoid(z1)
    ddesc = jnp.dot(dz1, W1t_ref[...], preferred_element_type=f32)
    drbf = ddesc[:, :P]
    dD = ddesc[:, P:]

    rio3 = jax.lax.broadcasted_iota(jnp.int32, (2 * P, 2), 0)
    cio3 = jax.lax.broadcasted_iota(jnp.int32, (2 * P, 2), 1)
    RED = (jax.lax.shift_right_logical(rio3, 4) == cio3).astype(f32)
    dsb = drbf * fc
    red = dotf(jnp.concatenate(
        [drbf * sb, dsb * (C0 * a * c_ar - sb)], axis=1), RED)
    dfc = dfc + red[:, 0:1]
    dr = red[:, 1:2] * rinv

    dDz = dotf(dD, RepMat) * SEL2
    dDatm = jnp.sum(dDz.reshape(B, K, KK), axis=1)
    S0 = None
    S1 = None
    for m in range(M):
        mu_m = -1.0 + 2.0 * m / (M - 1)
        Hm = dotf(dDatm[:, m * K:(m + 1) * K], RepMat) * Gms[m]
        S0 = Hm if S0 is None else S0 + Hm
        S1 = mu_m * Hm if S1 is None else S1 + mu_m * Hm
    dfc = dfc + lanes_to_pairs(dotf(S0 * maskL, SumL))
    dcos = (-2.0 * ETA) * (cos * S0 - S1) * wgt
    dus = []
    for d in range(3):
        du16 = dotf(dcos * uB[d], SumK) + dotf(dcos * uA[d], SumL)
        dus.append(lanes_to_pairs(du16))
    du = jnp.concatenate(dus, axis=1)

    dr = dr + dfc * (-0.5 * PI_C) * s1
    ddiff = dr * u + (du - jnp.sum(du * u, axis=1, keepdims=True) * u) * rinv

    fi_ref[...] = -jnp.sum(ddiff.reshape(B, K, 3), axis=1)
    g_ref[...] = jnp.concatenate(
        [ddiff, e, jnp.zeros((BK, 12), f32)], axis=1)


@functools.partial(jax.jit, static_argnames=("interpret",))
def _tc_call(x, xj16, transform_x, W1, b1, W2, b2, W3, b3, interpret=False):
    rep = lambda i: (0, 0)
    return pl.pallas_call(
        _tc_body,
        grid=(GRID,),
        in_specs=[
            pl.BlockSpec((B, 3), lambda i: (i, 0)),
            pl.BlockSpec((BK, 16), lambda i: (i, 0)),
            pl.BlockSpec((BK, 3), lambda i: (i, 0)),
            pl.BlockSpec((P + M, H), rep),
            pl.BlockSpec((1, H), rep),
            pl.BlockSpec((H, H), rep),
            pl.BlockSpec((1, H), rep),
            pl.BlockSpec((H, 1), rep),
            pl.BlockSpec((1, 1), rep),
            pl.BlockSpec((1, H), rep),
            pl.BlockSpec((H, H), rep),
            pl.BlockSpec((H, P + M), rep),
        ],
        out_specs=[
            pl.BlockSpec((B, 3), lambda i: (i, 0)),
            pl.BlockSpec((BK, 16), lambda i: (i, 0)),
        ],
        out_shape=[
            jax.ShapeDtypeStruct((NA, 3), jnp.float32),
            jax.ShapeDtypeStruct((E, 16), jnp.float32),
        ],
        interpret=interpret,
    )(x, xj16, transform_x, W1, b1.reshape(1, H), W2, b2.reshape(1, H), W3,
      b3.reshape(1, 1), W3.reshape(1, H), W2.T, W1.T)


NW = 32
CH = 100
NCH = E // (NW * CH)
RPS = NA // 16

def _sc_gather_body(x16_hbm, uj_hbm, out_hbm, idx_v, rows_v, sem):
    w = lax.axis_index("c") * 16 + lax.axis_index("s")
    pltpu.sync_copy(uj_hbm.at[w], idx_v)

    def body(j, carry):
        pltpu.async_copy(x16_hbm.at[idx_v.at[j]], rows_v.at[j], sem).wait()
        return carry

    lax.fori_loop(0, NCH, body, 0)
    pltpu.sync_copy(rows_v, out_hbm.at[w])


def _sc_scatter_body(g_hbm, uj_hbm, ind_hbm, zf_hbm, ze_hbm, outf_hbm,
                     oute_hbm, idxj_v, idxe_v, vals_v, accf_s, acce_s):
    c = lax.axis_index("c")
    s = lax.axis_index("s")
    w = c * 16 + s
    pltpu.sync_copy(zf_hbm.at[pl.ds(s * RPS, RPS)],
                    accf_s.at[pl.ds(s * RPS, RPS)])

    @pl.when(s == 0)
    def _():
        pltpu.sync_copy(ze_hbm, acce_s)

    plsc.subcore_barrier()
    pltpu.sync_copy(uj_hbm.at[w], idxj_v)
    pltpu.sync_copy(ind_hbm.at[w], idxe_v)
    pltpu.sync_copy(g_hbm.at[w], vals_v)

    def body(j, carry):
        pltpu.sync_copy(vals_v.at[j], accf_s.at[idxj_v.at[j]], add=True)
        pltpu.sync_copy(vals_v.at[j], acce_s.at[idxe_v.at[j]], add=True)
        return carry

    lax.fori_loop(0, NCH, body, 0)
    plsc.subcore_barrier()
    pltpu.sync_copy(accf_s.at[pl.ds(s * RPS, RPS)],
                    outf_hbm.at[c, pl.ds(s * RPS, RPS)])

    @pl.when(s == 0)
    def _():
        pltpu.sync_copy(acce_s, oute_hbm.at[c])


@functools.cache
def _sc_kernels():
    mesh = plsc.VectorSubcoreMesh(core_axis_name="c", subcore_axis_name="s")
    gather = pl.kernel(
        _sc_gather_body,
        out_type=jax.ShapeDtypeStruct((NW, NCH, CH, 16), jnp.float32),
        compiler_params=pltpu.CompilerParams(use_tc_tiling_on_sc=False),
        mesh=mesh,
        scratch_types=[
            pltpu.VMEM((NCH, CH), jnp.int32),
            pltpu.VMEM((NCH, CH, 16), jnp.float32),
            pltpu.SemaphoreType.DMA,
        ],
    )
    scatter = pl.kernel(
        _sc_scatter_body,
        out_type=(jax.ShapeDtypeStruct((2, NA, 16), jnp.float32),
                  jax.ShapeDtypeStruct((2, NC, 16), jnp.float32)),
        compiler_params=pltpu.CompilerParams(use_tc_tiling_on_sc=False),
        mesh=mesh,
        scratch_types=[
            pltpu.VMEM((NCH, CH), jnp.int32),
            pltpu.VMEM((NCH, CH), jnp.int32),
            pltpu.VMEM((NCH, CH, 16), jnp.float32),
            pltpu.VMEM_SHARED((NA, 16), jnp.float32),
            pltpu.VMEM_SHARED((NC, 16), jnp.float32),
        ],
    )
    return gather, scatter


def kernel(x, neighlist, transform_x, indices, atoms_per_structure, types,
           unique_i, unique_j, W1, b1, W2, b2, W3, b3):
    x16 = jnp.pad(x, ((0, 0), (0, 13)))
    ujc = unique_j.astype(jnp.int32).reshape(NW, NCH, CH)
    _sc_gather, _sc_scatter = _sc_kernels()
    xj16 = _sc_gather(x16, ujc).reshape(E, 16)
    fi, g16 = _tc_call(x, xj16, transform_x, W1, b1, W2, b2, W3, b3)
    outf, oute = _sc_scatter(
        g16.reshape(NW, NCH, CH, 16), ujc,
        indices.astype(jnp.int32).reshape(NW, NCH, CH),
        jnp.zeros((NA, 16), jnp.float32), jnp.zeros((NC, 16), jnp.float32))
    energy = oute[0, :, 3] + oute[1, :, 3]
    forces = fi + outf[0, :, :3] + outf[1, :, :3]
    return energy, forces
